# baseline (speedup 1.0000x reference)
import functools

import jax
import jax.numpy as jnp
from jax import lax
from jax.experimental import pallas as pl
from jax.experimental.pallas import tpu as pltpu
from jax.experimental.pallas import tpu_sc as plsc


def _gather_call(B, V, D):
    info = plsc.get_sparse_core_info()
    NC, NS, L = info.num_cores, info.num_subcores, info.num_lanes
    NW = NC * NS
    b_per_w = B // NW
    TR = D // 8
    TC = B // 128
    tc_per_w = TC // NW
    slab = tc_per_w * 8 * 128
    mesh = plsc.VectorSubcoreMesh(core_axis_name="c", subcore_axis_name="s")

    @functools.partial(
        pl.kernel,
        mesh=mesh,
        out_type=jax.ShapeDtypeStruct((TR * TC * 8 * 128,), jnp.float32),
        scratch_types=[
            pltpu.VMEM((b_per_w,), jnp.int32),
            pltpu.VMEM((b_per_w, D), jnp.float32),
            pltpu.VMEM((TR * slab,), jnp.float32),
            pltpu.SemaphoreType.DMA,
            [pltpu.SemaphoreType.DMA] * TR,
        ],
        compiler_params=pltpu.CompilerParams(
            use_tc_tiling_on_sc=False, needs_layout_passes=False),
    )
    def gather_kernel(table_hbm, idx_hbm, out_hbm, idx_v, rows_v, out_v,
                      gsem, ssems):
        wid = lax.axis_index("s") * NC + lax.axis_index("c")
        base = wid * b_per_w
        pltpu.sync_copy(idx_hbm.at[pl.ds(base, b_per_w)], idx_v)
        pltpu.async_copy(table_hbm.at[idx_v], rows_v, gsem).wait()

        iota = lax.iota(jnp.int32, L)

        @plsc.parallel_loop(0, b_per_w // L, 1, unroll=2)
        def transpose_chunk(jc):
            j16 = jc * L + iota
            off = (jc // 8) * 1024 + (jc % 8) * L
            for f in range(D):
                v = plsc.load_gather(rows_v, [j16, jnp.full((L,), f, jnp.int32)])
                out_v[pl.ds((f // 8) * slab + (f % 8) * 128 + off, L)] = v

        tc0 = wid * tc_per_w
        stores = [
            pltpu.async_copy(
                out_v.at[pl.ds(tr * slab, slab)],
                out_hbm.at[pl.ds(tr * TC * 1024 + tc0 * 1024, slab)],
                ssems[tr])
            for tr in range(TR)
        ]
        for cp in stores:
            cp.wait()

    return gather_kernel


def kernel(labels, emb_table):
    B, = labels.shape
    V, D = emb_table.shape
    flat = _gather_call(B, V, D)(emb_table, labels.astype(jnp.int32))
    return (flat.reshape(D // 8, B // 128, 8, 128)
            .transpose(1, 3, 0, 2).reshape(B, D))


# speedup vs baseline: 1.0982x; 1.0982x over previous
import functools

import jax
import jax.numpy as jnp
from jax import lax
from jax.experimental import pallas as pl
from jax.experimental.pallas import tpu as pltpu
from jax.experimental.pallas import tpu_sc as plsc


def _gather_call(B, V, D):
    info = plsc.get_sparse_core_info()
    NC, NS, L = info.num_cores, info.num_subcores, info.num_lanes
    NW = NC * NS
    b_per_w = B // NW
    TR = D // 8
    TC = B // 128
    tc_per_w = TC // NW
    slab = tc_per_w * 8 * 128
    mesh = plsc.VectorSubcoreMesh(core_axis_name="c", subcore_axis_name="s")

    @functools.partial(
        pl.kernel,
        mesh=mesh,
        out_type=jax.ShapeDtypeStruct((TR * TC * 8 * 128,), jnp.float32),
        scratch_types=[
            pltpu.VMEM((b_per_w,), jnp.int32),
            pltpu.VMEM((b_per_w, D), jnp.float32),
            pltpu.VMEM((TR * slab,), jnp.float32),
            pltpu.SemaphoreType.DMA,
            [pltpu.SemaphoreType.DMA] * TR,
        ],
        compiler_params=pltpu.CompilerParams(
            use_tc_tiling_on_sc=False, needs_layout_passes=False,
            disable_bounds_checks=True),
    )
    def gather_kernel(table_hbm, idx_hbm, out_hbm, idx_v, rows_v, out_v,
                      gsem, ssems):
        wid = lax.axis_index("s") * NC + lax.axis_index("c")
        base = wid * b_per_w
        pltpu.sync_copy(idx_hbm.at[pl.ds(base, b_per_w)], idx_v)
        pltpu.async_copy(table_hbm.at[idx_v], rows_v, gsem).wait()

        iota = lax.iota(jnp.int32, L)
        addr_base = [
            (2 * k + iota // 8) * slab + (iota % 8) * 128
            for k in range(D // L)
        ]

        @plsc.parallel_loop(0, b_per_w, 1, unroll=4)
        def transpose_row(j):
            off = (j // 128) * 1024 + j % 128
            for k in range(D // L):
                v = rows_v[j, pl.ds(k * L, L)]
                plsc.store_scatter(out_v, [addr_base[k] + off], v)

        tc0 = wid * tc_per_w
        stores = [
            pltpu.async_copy(
                out_v.at[pl.ds(tr * slab, slab)],
                out_hbm.at[pl.ds(tr * TC * 1024 + tc0 * 1024, slab)],
                ssems[tr])
            for tr in range(TR)
        ]
        for cp in stores:
            cp.wait()

    return gather_kernel


def kernel(labels, emb_table):
    B, = labels.shape
    V, D = emb_table.shape
    flat = _gather_call(B, V, D)(emb_table, labels.astype(jnp.int32))
    return (flat.reshape(D // 8, B // 128, 8, 128)
            .transpose(1, 3, 0, 2).reshape(B, D))


# speedup vs baseline: 1.2321x; 1.1220x over previous
import functools

import jax
import jax.numpy as jnp
from jax import lax
from jax.experimental import pallas as pl
from jax.experimental.pallas import tpu as pltpu
from jax.experimental.pallas import tpu_sc as plsc


def _gather_call(B, V, D):
    info = plsc.get_sparse_core_info()
    NC, NS, L = info.num_cores, info.num_subcores, info.num_lanes
    NW = NC * NS
    b_per_w = B // NW
    TR = D // 8
    TC = B // 128
    tc_per_w = TC // NW
    slab = tc_per_w * 8 * 128
    n_jc = b_per_w // L
    mesh = plsc.VectorSubcoreMesh(core_axis_name="c", subcore_axis_name="s")

    @functools.partial(
        pl.kernel,
        mesh=mesh,
        out_type=jax.ShapeDtypeStruct((TR * TC * 8 * 128,), jnp.float32),
        scratch_types=[
            pltpu.VMEM((b_per_w,), jnp.int32),
            pltpu.VMEM((V * D,), jnp.float32),
            pltpu.VMEM((TR * slab,), jnp.float32),
            [pltpu.SemaphoreType.DMA] * TR,
        ],
        compiler_params=pltpu.CompilerParams(
            use_tc_tiling_on_sc=False, needs_layout_passes=False,
            disable_bounds_checks=True),
    )
    def gather_kernel(tablet_hbm, idx_hbm, out_hbm, idx_v, tab_v, out_v,
                      ssems):
        wid = lax.axis_index("s") * NC + lax.axis_index("c")
        base = wid * b_per_w
        pltpu.sync_copy(idx_hbm.at[pl.ds(base, b_per_w)], idx_v)
        pltpu.sync_copy(tablet_hbm, tab_v)

        stepv = jnp.full((L,), V, jnp.int32)

        @plsc.parallel_loop(0, n_jc, 1, unroll=2)
        def gather_chunk(jc):
            off = (jc // 8) * 1024 + (jc % 8) * L
            addr = idx_v[pl.ds(jc * L, L)]
            for f in range(D):
                v = plsc.load_gather(tab_v, [addr])
                out_v[pl.ds((f // 8) * slab + (f % 8) * 128 + off, L)] = v
                addr = addr + stepv

        tc0 = wid * tc_per_w
        stores = [
            pltpu.async_copy(
                out_v.at[pl.ds(tr * slab, slab)],
                out_hbm.at[pl.ds(tr * TC * 1024 + tc0 * 1024, slab)],
                ssems[tr])
            for tr in range(TR)
        ]
        for cp in stores:
            cp.wait()

    return gather_kernel


def kernel(labels, emb_table):
    B, = labels.shape
    V, D = emb_table.shape
    tablet = emb_table.T.reshape(-1)
    flat = _gather_call(B, V, D)(tablet, labels.astype(jnp.int32))
    return (flat.reshape(D // 8, B // 128, 8, 128)
            .transpose(1, 3, 0, 2).reshape(B, D))


# speedup vs baseline: 1.3594x; 1.1033x over previous
import functools

import jax
import jax.numpy as jnp
from jax import lax
from jax.experimental import pallas as pl
from jax.experimental.pallas import tpu as pltpu
from jax.experimental.pallas import tpu_sc as plsc


def _gather_call(B, V, D):
    info = plsc.get_sparse_core_info()
    NC, NS, L = info.num_cores, info.num_subcores, info.num_lanes
    NW = NC * NS
    b_per_w = B // NW
    TR = D // 8
    TC = B // 128
    tc_per_w = TC // NW
    slab = tc_per_w * 8 * 128
    n_jc = b_per_w // L
    mesh = plsc.VectorSubcoreMesh(core_axis_name="c", subcore_axis_name="s")

    @functools.partial(
        pl.kernel,
        mesh=mesh,
        out_type=jax.ShapeDtypeStruct((TR * TC * 8 * 128,), jnp.float32),
        scratch_types=[
            pltpu.VMEM((b_per_w,), jnp.int32),
            pltpu.VMEM((V * D,), jnp.float32),
            pltpu.VMEM((TR * slab,), jnp.float32),
            [pltpu.SemaphoreType.DMA] * TR,
            [pltpu.SemaphoreType.DMA] * TR,
        ],
        compiler_params=pltpu.CompilerParams(
            use_tc_tiling_on_sc=False, needs_layout_passes=False,
            disable_bounds_checks=True),
    )
    def gather_kernel(tablet_hbm, idx_hbm, out_hbm, idx_v, tab_v, out_v,
                      gsems, ssems):
        wid = lax.axis_index("s") * NC + lax.axis_index("c")
        base = wid * b_per_w
        stages = [
            pltpu.async_copy(
                tablet_hbm.at[pl.ds(tr * 8 * V, 8 * V)],
                tab_v.at[pl.ds(tr * 8 * V, 8 * V)], gsems[tr])
            for tr in range(TR)
        ]
        pltpu.sync_copy(idx_hbm.at[pl.ds(base, b_per_w)], idx_v)

        stepv = jnp.full((L,), V, jnp.int32)
        tc0 = wid * tc_per_w
        stores = []
        for tr in range(TR):
            stages[tr].wait()
            trbase = jnp.full((L,), tr * 8 * V, jnp.int32)

            @plsc.parallel_loop(0, n_jc, 1, unroll=2)
            def gather_chunk(jc, tr=tr, trbase=trbase):
                off = (jc // 8) * 1024 + (jc % 8) * L
                addr = idx_v[pl.ds(jc * L, L)] + trbase
                for r in range(8):
                    v = plsc.load_gather(tab_v, [addr])
                    out_v[pl.ds(tr * slab + r * 128 + off, L)] = v
                    addr = addr + stepv

            stores.append(pltpu.async_copy(
                out_v.at[pl.ds(tr * slab, slab)],
                out_hbm.at[pl.ds(tr * TC * 1024 + tc0 * 1024, slab)],
                ssems[tr]))
        for cp in stores:
            cp.wait()

    return gather_kernel


def kernel(labels, emb_table):
    B, = labels.shape
    V, D = emb_table.shape
    tablet = emb_table.T.reshape(-1)
    flat = _gather_call(B, V, D)(tablet, labels.astype(jnp.int32))
    return (flat.reshape(D // 8, B // 128, 8, 128)
            .transpose(1, 3, 0, 2).reshape(B, D))
